# baseline (speedup 1.0000x reference)
import functools

import jax
import jax.numpy as jnp
from jax import lax
from jax.experimental import pallas as pl
from jax.experimental.pallas import tpu as pltpu
from jax.experimental.pallas import tpu_sc as plsc

N_NODES = 10000
NODE_SIZE = 128
NUM_GRAPHS = 128
GLOBAL_SIZE = 64
NC = 2
NS = 16
L = 16
C = 320
SEG_ROWS = 136
TRASH = NUM_GRAPHS
ZROWS = 16
CHUNKS = (128, 32, 128, 32)
HBLK = 512

_SELU_SCALE = 1.0507009873554805
_SELU_ALPHA = 1.6732632423543772


def _sc_segsum(x, batch):
    mesh = plsc.VectorSubcoreMesh(core_axis_name="c", subcore_axis_name="s")

    @functools.partial(
        pl.kernel,
        mesh=mesh,
        out_type=jax.ShapeDtypeStruct((NC * SEG_ROWS, NODE_SIZE), jnp.float32),
        scratch_types=[
            pltpu.VMEM((C, NODE_SIZE), jnp.float32),
            pltpu.VMEM((C,), jnp.int32),
            pltpu.VMEM((len(CHUNKS), 128), jnp.int32),
            pltpu.VMEM((ZROWS, NODE_SIZE), jnp.float32),
            pltpu.VMEM_SHARED((SEG_ROWS, NODE_SIZE), jnp.float32),
            pltpu.SemaphoreType.DMA,
            pltpu.SemaphoreType.DMA,
            pltpu.SemaphoreType.DMA,
            pltpu.SemaphoreType.DMA,
        ],
    )
    def seg_kernel(x_hbm, b_hbm, acc_out, xv, bv, idx2, zbuf, acc_sh,
                   semx0, semx1, semb, sems):
        cid = lax.axis_index("c")
        sid = lax.axis_index("s")
        wid = sid * NC + cid
        thr = wid * C
        base = jnp.minimum(thr, N_NODES - C)

        H = C // 2
        hx0 = pltpu.async_copy(x_hbm.at[pl.ds(base, H), :],
                               xv.at[pl.ds(0, H), :], semx0)
        hb = pltpu.async_copy(b_hbm.at[pl.ds(base, C)], bv, semb)
        hx1 = pltpu.async_copy(x_hbm.at[pl.ds(base + H, H), :],
                               xv.at[pl.ds(H, H), :], semx1)

        zero = jnp.zeros((L,), jnp.float32)
        for i in range(ZROWS):
            for j in range(NODE_SIZE // L):
                zbuf[i, pl.ds(j * L, L)] = zero
        zbase = jnp.minimum(sid * ZROWS, SEG_ROWS - ZROWS)
        pltpu.sync_copy(zbuf, acc_sh.at[pl.ds(zbase, ZROWS)])

        hb.wait()
        iota = lax.iota(jnp.int32, L)
        bounds = []
        off = 0
        for csz in CHUNKS:
            bounds.append((off, off + csz))
            off += csz
        for k in range(C // L):
            ids = bv[pl.ds(k * L, L)]
            gpos = base + k * L + iota
            sel = jnp.where(gpos >= thr, ids, TRASH)
            p = k * L
            ci = next(i for i, (lo, hi) in enumerate(bounds) if lo <= p < hi)
            idx2[ci, pl.ds(p - bounds[ci][0], L)] = sel

        hx0.wait()
        plsc.subcore_barrier()

        handles = []

        def scatter(off, csz, ci):
            handles.append(pltpu.async_copy(
                xv.at[pl.ds(off, csz), :],
                acc_sh.at[idx2.at[ci, pl.ds(0, csz)]], sems, add=True))

        scatter(0, 128, 0)
        scatter(128, 32, 1)
        hx1.wait()
        scatter(160, 128, 2)
        scatter(288, 32, 3)
        for h in handles:
            h.wait()

        plsc.subcore_barrier()

        @pl.when(sid == 0)
        def _():
            pltpu.sync_copy(acc_sh, acc_out.at[pl.ds(cid * SEG_ROWS, SEG_ROWS)])

    return seg_kernel(x, batch)


def _tc_hist(b_ref, cnt_ref):
    gid = lax.broadcasted_iota(jnp.int32, (NUM_GRAPHS, 1), 0)
    cnt = jnp.zeros((NUM_GRAPHS, 1), jnp.float32)
    for k in range(0, N_NODES, HBLK):
        blk = b_ref[:, k:min(k + HBLK, N_NODES)]
        eq = (blk == gid).astype(jnp.float32)
        cnt = cnt + jnp.sum(eq, axis=1, keepdims=True)
    cnt_ref[:] = jnp.maximum(cnt, 1.0)


def _tc_mlp(acc_ref, cnt_ref, u_ref, w1_ref, b1_ref, w2_ref, b2_ref, out_ref):
    s = (acc_ref[:NUM_GRAPHS, :]
         + acc_ref[SEG_ROWS:SEG_ROWS + NUM_GRAPHS, :])
    mean = s / cnt_ref[:]
    h = (jnp.dot(u_ref[:], w1_ref[:GLOBAL_SIZE, :],
                 preferred_element_type=jnp.float32)
         + jnp.dot(mean, w1_ref[GLOBAL_SIZE:, :],
                   preferred_element_type=jnp.float32)
         + b1_ref[:])
    h = _SELU_SCALE * jnp.where(h > 0, h, _SELU_ALPHA * (jnp.exp(h) - 1.0))
    out_ref[:] = (jnp.dot(h, w2_ref[:], preferred_element_type=jnp.float32)
                  + b2_ref[:])


def kernel(x, edge_index, edge_attr, u, batch, W1, b1, W2, b2):
    acc = _sc_segsum(x, batch)
    cnt = pl.pallas_call(
        _tc_hist,
        out_shape=jax.ShapeDtypeStruct((NUM_GRAPHS, 1), jnp.float32),
    )(batch.reshape(1, N_NODES))
    return pl.pallas_call(
        _tc_mlp,
        out_shape=jax.ShapeDtypeStruct((NUM_GRAPHS, W2.shape[1]), jnp.float32),
    )(acc, cnt, u, W1, b1.reshape(1, -1), W2, b2.reshape(1, -1))


# speedup vs baseline: 1.0609x; 1.0609x over previous
import functools

import jax
import jax.numpy as jnp
from jax import lax
from jax.experimental import pallas as pl
from jax.experimental.pallas import tpu as pltpu
from jax.experimental.pallas import tpu_sc as plsc

N_NODES = 10000
NODE_SIZE = 128
NUM_GRAPHS = 128
GLOBAL_SIZE = 64
NC = 2
NS = 16
L = 16
C = 320
SEG_ROWS = 136
TRASH = NUM_GRAPHS
ZROWS = 16
CHUNKS = (128, 32, 128, 32)
HBLK = 512

_SELU_SCALE = 1.0507009873554805
_SELU_ALPHA = 1.6732632423543772


def _sc_segsum(x, batch):
    mesh = plsc.VectorSubcoreMesh(core_axis_name="c", subcore_axis_name="s")

    @functools.partial(
        pl.kernel,
        mesh=mesh,
        out_type=jax.ShapeDtypeStruct((NC * SEG_ROWS, NODE_SIZE), jnp.float32),
        scratch_types=[
            pltpu.VMEM((C, NODE_SIZE), jnp.float32),
            pltpu.VMEM((C,), jnp.int32),
            pltpu.VMEM((len(CHUNKS), 128), jnp.int32),
            pltpu.VMEM((ZROWS, NODE_SIZE), jnp.float32),
            pltpu.VMEM_SHARED((SEG_ROWS, NODE_SIZE), jnp.float32),
            pltpu.SemaphoreType.DMA,
            pltpu.SemaphoreType.DMA,
            pltpu.SemaphoreType.DMA,
            pltpu.SemaphoreType.DMA,
        ],
    )
    def seg_kernel(x_hbm, b_hbm, acc_out, xv, bv, idx2, zbuf, acc_sh,
                   semx0, semx1, semb, sems):
        cid = lax.axis_index("c")
        sid = lax.axis_index("s")
        wid = sid * NC + cid
        thr = wid * C
        base = jnp.minimum(thr, N_NODES - C)

        H = C // 2
        hx0 = pltpu.async_copy(x_hbm.at[pl.ds(base, H), :],
                               xv.at[pl.ds(0, H), :], semx0)
        hb = pltpu.async_copy(b_hbm.at[pl.ds(base, C)], bv, semb)
        hx1 = pltpu.async_copy(x_hbm.at[pl.ds(base + H, H), :],
                               xv.at[pl.ds(H, H), :], semx1)

        zero = jnp.zeros((L,), jnp.float32)
        for i in range(ZROWS):
            for j in range(NODE_SIZE // L):
                zbuf[i, pl.ds(j * L, L)] = zero
        zbase = jnp.minimum(sid * ZROWS, SEG_ROWS - ZROWS)
        pltpu.sync_copy(zbuf, acc_sh.at[pl.ds(zbase, ZROWS)])

        hb.wait()
        iota = lax.iota(jnp.int32, L)
        bounds = []
        off = 0
        for csz in CHUNKS:
            bounds.append((off, off + csz))
            off += csz
        for k in range(C // L):
            ids = bv[pl.ds(k * L, L)]
            gpos = base + k * L + iota
            sel = jnp.where(gpos >= thr, ids, TRASH)
            p = k * L
            ci = next(i for i, (lo, hi) in enumerate(bounds) if lo <= p < hi)
            idx2[ci, pl.ds(p - bounds[ci][0], L)] = sel

        hx0.wait()
        plsc.subcore_barrier()

        handles = []

        def scatter(off, csz, ci):
            handles.append(pltpu.async_copy(
                xv.at[pl.ds(off, csz), :],
                acc_sh.at[idx2.at[ci, pl.ds(0, csz)]], sems, add=True))

        scatter(0, 128, 0)
        scatter(128, 32, 1)
        hx1.wait()
        scatter(160, 128, 2)
        scatter(288, 32, 3)
        for h in handles:
            h.wait()

        plsc.subcore_barrier()

        @pl.when(sid == 0)
        def _():
            pltpu.sync_copy(acc_sh, acc_out.at[pl.ds(cid * SEG_ROWS, SEG_ROWS)])

    return seg_kernel(x, batch)


def _tc_hist(b_ref, cnt_ref):
    gid = lax.broadcasted_iota(jnp.int32, (NUM_GRAPHS, 1), 0)
    cnt = jnp.zeros((NUM_GRAPHS, 1), jnp.float32)
    for k in range(0, N_NODES, HBLK):
        blk = b_ref[:, k:min(k + HBLK, N_NODES)]
        eq = (blk == gid).astype(jnp.float32)
        cnt = cnt + jnp.sum(eq, axis=1, keepdims=True)
    cnt_ref[:] = jnp.maximum(cnt, 1.0)


def _tc_mlp(acc_ref, cnt_ref, u_ref, w1_ref, b1_ref, w2_ref, b2_ref, out_ref):
    s = (acc_ref[:NUM_GRAPHS, :]
         + acc_ref[SEG_ROWS:SEG_ROWS + NUM_GRAPHS, :])
    mean = s / cnt_ref[:]
    h = (jnp.dot(u_ref[:], w1_ref[:GLOBAL_SIZE, :],
                 preferred_element_type=jnp.float32)
         + jnp.dot(mean, w1_ref[GLOBAL_SIZE:, :],
                   preferred_element_type=jnp.float32)
         + b1_ref[:])
    h = _SELU_SCALE * jnp.where(h > 0, h, _SELU_ALPHA * (jnp.exp(h) - 1.0))
    out_t = lax.dot_general(w2_ref[:], h, (((0,), (1,)), ((), ())),
                            preferred_element_type=jnp.float32)
    out_ref[:] = out_t + b2_ref[:]


def kernel(x, edge_index, edge_attr, u, batch, W1, b1, W2, b2):
    acc = _sc_segsum(x, batch)
    cnt = pl.pallas_call(
        _tc_hist,
        out_shape=jax.ShapeDtypeStruct((NUM_GRAPHS, 1), jnp.float32),
    )(batch.reshape(1, N_NODES))
    out_t = pl.pallas_call(
        _tc_mlp,
        out_shape=jax.ShapeDtypeStruct((W2.shape[1], NUM_GRAPHS), jnp.float32),
    )(acc, cnt, u, W1, b1.reshape(1, -1), W2, b2.reshape(-1, 1))
    return out_t.T
